# Initial kernel scaffold; baseline (speedup 1.0000x reference)
import jax
import jax.numpy as jnp
from jax.experimental import pallas as pl


def kernel(x, edge_index, W1, b1, W2, b2):
    raise NotImplementedError("write your pallas kernel here")


# speedup vs baseline: 20.2559x; 20.2559x over previous
import functools

import jax
import jax.numpy as jnp
from jax import lax
from jax.experimental import pallas as pl
from jax.experimental.pallas import tpu as pltpu
from jax.experimental.pallas import tpu_sc as plsc

N_NODES = 10000
N_EDGES = 320000
D = 128

NC = 2
NS = 16
NW = NC * NS

CB = 128
NCH = 158
E_PAD = NS * NCH * CB
NPAD = 10240
HALF = NPAD // 2
RPT = HALF // NS

BR = 1024
GRID = NPAD // BR


def _sc_degree_body(dst_hbm, out_hbm, dst_v, buf_v, acc_s):
    c = lax.axis_index("c")
    s = lax.axis_index("s")
    wid = c * NS + s
    base = s * RPT

    pltpu.sync_copy(dst_hbm.at[wid], dst_v)

    @pl.loop(0, 64)
    def _(i):
        for j in range(D // 16):
            buf_v[i, pl.ds(j * 16, 16)] = jnp.zeros((16,), jnp.float32)

    for k in range(RPT // 64):
        pltpu.sync_copy(buf_v.at[pl.ds(0, 64)],
                        acc_s.at[pl.ds(base + k * 64, 64)])
    plsc.subcore_barrier()

    @pl.loop(0, CB)
    def _(i):
        for j in range(D // 16):
            buf_v[i, pl.ds(j * 16, 16)] = jnp.full((16,), 1.0, jnp.float32)

    @pl.loop(0, NCH)
    def _(j):
        idx = plsc.Indices(dst_v.at[j], ignored_value=-1)
        pltpu.sync_copy(buf_v, acc_s.at[idx], add=True)

    plsc.subcore_barrier()

    for k in range(RPT // 64):
        r0 = base + k * 64
        pltpu.sync_copy(acc_s.at[pl.ds(r0, 64)], buf_v.at[pl.ds(0, 64)])
        pltpu.sync_copy(buf_v.at[pl.ds(0, 64)], out_hbm.at[c].at[pl.ds(r0, 64)])


def _sc_scatter_body(g_hbm, src_hbm, dst_hbm, out_hbm,
                     src_v, dst_v, buf0, buf1, acc_s, sem0, sem1):
    c = lax.axis_index("c")
    s = lax.axis_index("s")
    wid = c * NS + s
    base = s * RPT

    pltpu.sync_copy(src_hbm.at[wid], src_v)
    pltpu.sync_copy(dst_hbm.at[wid], dst_v)

    @pl.loop(0, 64)
    def _(i):
        for j in range(D // 16):
            buf0[i, pl.ds(j * 16, 16)] = jnp.zeros((16,), jnp.float32)

    for k in range(RPT // 64):
        pltpu.sync_copy(buf0.at[pl.ds(0, 64)],
                        acc_s.at[pl.ds(base + k * 64, 64)])
    plsc.subcore_barrier()

    def _gather(j, buf, sem):
        idx = plsc.Indices(src_v.at[j], ignored_value=-1)
        return pltpu.async_copy(g_hbm.at[idx], buf, sem)

    def _gather_wait(j, buf, sem):
        idx = plsc.Indices(src_v.at[j], ignored_value=-1)
        pltpu.make_async_copy(g_hbm.at[idx], buf, sem).wait()

    def _scatter(j, buf):
        idx = plsc.Indices(dst_v.at[j], ignored_value=-1)
        pltpu.sync_copy(buf, acc_s.at[idx], add=True)

    _gather(0, buf0, sem0)

    @pl.loop(0, NCH // 2)
    def _(jj):
        j0 = jj * 2
        _gather(j0 + 1, buf1, sem1)
        _gather_wait(j0, buf0, sem0)
        _scatter(j0, buf0)

        @pl.when(jj < NCH // 2 - 1)
        def _():
            _gather(j0 + 2, buf0, sem0)

        _gather_wait(j0 + 1, buf1, sem1)
        _scatter(j0 + 1, buf1)

    plsc.subcore_barrier()

    for k in range(RPT // 64):
        r0 = base + k * 64
        pltpu.sync_copy(acc_s.at[pl.ds(r0, 64)], buf0.at[pl.ds(0, 64)])
        pltpu.sync_copy(buf0.at[pl.ds(0, 64)], out_hbm.at[c].at[pl.ds(r0, 64)])


@functools.cache
def _sc_kernels():
    mesh = plsc.VectorSubcoreMesh(core_axis_name="c", subcore_axis_name="s",
                                  num_cores=NC, num_subcores=NS)
    sc_degree = pl.kernel(
        _sc_degree_body,
        out_type=jax.ShapeDtypeStruct((NC, HALF, D), jnp.float32),
        mesh=mesh,
        scratch_types=[
            pltpu.VMEM((NCH, CB), jnp.int32),
            pltpu.VMEM((CB, D), jnp.float32),
            pltpu.VMEM_SHARED((HALF, D), jnp.float32),
        ],
    )
    sc_scatter = pl.kernel(
        _sc_scatter_body,
        out_type=jax.ShapeDtypeStruct((NC, HALF, D), jnp.float32),
        mesh=mesh,
        scratch_types=[
            pltpu.VMEM((NCH, CB), jnp.int32),
            pltpu.VMEM((NCH, CB), jnp.int32),
            pltpu.VMEM((CB, D), jnp.float32),
            pltpu.VMEM((CB, D), jnp.float32),
            pltpu.VMEM_SHARED((HALF, D), jnp.float32),
            pltpu.SemaphoreType.DMA,
            pltpu.SemaphoreType.DMA,
        ],
    )
    return sc_degree, sc_scatter


def _mm_body(x_ref, w_ref, o_ref):
    o_ref[...] = jnp.dot(x_ref[...], w_ref[...],
                         preferred_element_type=jnp.float32)


def _scale_body(h_ref, dg_ref, o_ref):
    deg = dg_ref[:, 0:1] + 1.0
    o_ref[...] = h_ref[...] * lax.rsqrt(deg)


def _mid_body(s_ref, g_ref, dg_ref, w_ref, b_ref, o_ref):
    dinv = lax.rsqrt(dg_ref[:, 0:1] + 1.0)
    z = dinv * (s_ref[...] + g_ref[...]) + b_ref[...]
    r = jnp.maximum(z, 0.0)
    o_ref[...] = jnp.dot(r, w_ref[...],
                         preferred_element_type=jnp.float32) * dinv


def _final_body(s_ref, g_ref, dg_ref, b_ref, o_ref):
    dinv = lax.rsqrt(dg_ref[:, 0:1] + 1.0)
    z = dinv * (s_ref[...] + g_ref[...]) + b_ref[...]
    m = jnp.max(z, axis=1, keepdims=True)
    e = jnp.exp(z - m)
    o_ref[...] = (z - m) - jnp.log(jnp.sum(e, axis=1, keepdims=True))


_row_spec = pl.BlockSpec((BR, D), lambda i: (i, 0))
_deg_spec = pl.BlockSpec((BR, D), lambda i: (i, 0))
_w_spec = pl.BlockSpec((D, D), lambda i: (0, 0))
_b_spec = pl.BlockSpec((1, D), lambda i: (0, 0))
_out_struct = jax.ShapeDtypeStruct((NPAD, D), jnp.float32)

_tc_mm = pl.pallas_call(
    _mm_body, grid=(GRID,),
    in_specs=[_row_spec, _w_spec], out_specs=_row_spec,
    out_shape=_out_struct)

_tc_scale = pl.pallas_call(
    _scale_body, grid=(GRID,),
    in_specs=[_row_spec, _deg_spec], out_specs=_row_spec,
    out_shape=_out_struct)

_tc_mid = pl.pallas_call(
    _mid_body, grid=(GRID,),
    in_specs=[_row_spec, _row_spec, _deg_spec, _w_spec, _b_spec],
    out_specs=_row_spec, out_shape=_out_struct)

_tc_final = pl.pallas_call(
    _final_body, grid=(GRID,),
    in_specs=[_row_spec, _row_spec, _deg_spec, _b_spec],
    out_specs=_row_spec, out_shape=_out_struct)


def kernel(x, edge_index, W1, b1, W2, b2):
    src = edge_index[0].astype(jnp.int32)
    dst = edge_index[1].astype(jnp.int32)

    fill = jnp.full((E_PAD - N_EDGES,), -1, jnp.int32)
    src_p = jnp.concatenate([src, fill])
    dst_p = jnp.concatenate([dst, fill])
    in_a = (dst_p >= 0) & (dst_p < HALF)
    in_b = dst_p >= HALF
    neg1 = jnp.int32(-1)

    def slab(a):
        return a.reshape(NCH, NS, CB).transpose(1, 0, 2)

    src_slabs = jnp.stack([slab(jnp.where(in_a, src_p, neg1)),
                           slab(jnp.where(in_b, src_p, neg1))])
    dst_slabs = jnp.stack([slab(jnp.where(in_a, dst_p, neg1)),
                           slab(jnp.where(in_b, dst_p - HALF, neg1))])
    src_slabs = src_slabs.reshape(NW, NCH, CB)
    dst_slabs = dst_slabs.reshape(NW, NCH, CB)

    x_pad = jnp.zeros((NPAD, D), jnp.float32).at[:N_NODES].set(x)
    b1r = b1.reshape(1, D)
    b2r = b2.reshape(1, D)

    sc_degree, sc_scatter = _sc_kernels()
    deg = sc_degree(dst_slabs).reshape(NPAD, D)

    h1 = _tc_mm(x_pad, W1)
    g1 = _tc_scale(h1, deg)
    s1 = sc_scatter(g1, src_slabs, dst_slabs).reshape(NPAD, D)
    g2 = _tc_mid(s1, g1, deg, W2, b1r)
    s2 = sc_scatter(g2, src_slabs, dst_slabs).reshape(NPAD, D)
    out = _tc_final(s2, g2, deg, b2r)
    return out[:N_NODES]


# speedup vs baseline: 22.1373x; 1.0929x over previous
import functools

import jax
import jax.numpy as jnp
from jax import lax
from jax.experimental import pallas as pl
from jax.experimental.pallas import tpu as pltpu
from jax.experimental.pallas import tpu_sc as plsc

N_NODES = 10000
N_EDGES = 320000
D = 128

NC = 2
NS = 16
NW = NC * NS

CB = 128
NCH = 159
E_PAD = NS * NCH * CB
NPAD = 10240
HALF = NPAD // 2
RPT = HALF // NS

BR = 1024
GRID = NPAD // BR


def _sc_degree_body(dst_hbm, out_hbm, dst_v, buf_v, acc_s):
    c = lax.axis_index("c")
    s = lax.axis_index("s")
    wid = c * NS + s
    base = s * RPT

    pltpu.sync_copy(dst_hbm.at[wid], dst_v)

    @pl.loop(0, 64)
    def _(i):
        for j in range(D // 16):
            buf_v[i, pl.ds(j * 16, 16)] = jnp.zeros((16,), jnp.float32)

    for k in range(RPT // 64):
        pltpu.sync_copy(buf_v.at[pl.ds(0, 64)],
                        acc_s.at[pl.ds(base + k * 64, 64)])
    plsc.subcore_barrier()

    @pl.loop(0, CB)
    def _(i):
        for j in range(D // 16):
            buf_v[i, pl.ds(j * 16, 16)] = jnp.full((16,), 1.0, jnp.float32)

    @pl.loop(0, NCH)
    def _(j):
        idx = plsc.Indices(dst_v.at[j], ignored_value=-1)
        pltpu.sync_copy(buf_v, acc_s.at[idx], add=True)

    plsc.subcore_barrier()

    for k in range(RPT // 64):
        r0 = base + k * 64
        pltpu.sync_copy(acc_s.at[pl.ds(r0, 64)], buf_v.at[pl.ds(0, 64)])
        pltpu.sync_copy(buf_v.at[pl.ds(0, 64)], out_hbm.at[c].at[pl.ds(r0, 64)])


def _sc_scatter_body(g_hbm, src_hbm, dst_hbm, out_hbm,
                     src_v, dst_v, b0, b1, b2, acc_s,
                     g0, g1, g2, s0, s1, s2):
    c = lax.axis_index("c")
    s = lax.axis_index("s")
    wid = c * NS + s
    base = s * RPT
    bufs = [b0, b1, b2]
    gsem = [g0, g1, g2]
    ssem = [s0, s1, s2]

    pltpu.sync_copy(src_hbm.at[wid], src_v)
    pltpu.sync_copy(dst_hbm.at[wid], dst_v)

    def _gather(j, b):
        idx = plsc.Indices(src_v.at[j], ignored_value=-1)
        pltpu.async_copy(g_hbm.at[idx], bufs[b], gsem[b])

    def _gather_wait(j, b):
        idx = plsc.Indices(src_v.at[j], ignored_value=-1)
        pltpu.make_async_copy(g_hbm.at[idx], bufs[b], gsem[b]).wait()

    def _scatter(j, b):
        idx = plsc.Indices(dst_v.at[j], ignored_value=-1)
        pltpu.async_copy(bufs[b], acc_s.at[idx], ssem[b], add=True)

    def _scatter_wait(j, b):
        idx = plsc.Indices(dst_v.at[j], ignored_value=-1)
        pltpu.make_async_copy(bufs[b], acc_s.at[idx], ssem[b]).wait()

    for b in range(2):
        _gather(b, b)

    @pl.loop(0, 64)
    def _(i):
        for j in range(D // 16):
            b2[i, pl.ds(j * 16, 16)] = jnp.zeros((16,), jnp.float32)

    for k in range(RPT // 64):
        pltpu.sync_copy(b2.at[pl.ds(0, 64)],
                        acc_s.at[pl.ds(base + k * 64, 64)])
    plsc.subcore_barrier()

    @pl.loop(0, NCH // 3)
    def _(jj):
        for b in range(3):
            j = jj * 3 + b
            _gather_wait(j, b)
            _scatter(j, b)
            t = (b + 2) % 3
            jn = j + 2

            @pl.when(jn < NCH)
            def _():
                prev = jn - 3

                @pl.when(prev >= 0)
                def _():
                    _scatter_wait(prev, t)

                _gather(jn, t)

    for b in range(3):
        _scatter_wait(NCH - 3 + b, b)
    plsc.subcore_barrier()

    for k in range(RPT // 64):
        r0 = base + k * 64
        pltpu.sync_copy(acc_s.at[pl.ds(r0, 64)], b0.at[pl.ds(0, 64)])
        pltpu.sync_copy(b0.at[pl.ds(0, 64)], out_hbm.at[c].at[pl.ds(r0, 64)])


@functools.cache
def _sc_kernels():
    mesh = plsc.VectorSubcoreMesh(core_axis_name="c", subcore_axis_name="s",
                                  num_cores=NC, num_subcores=NS)
    sc_degree = pl.kernel(
        _sc_degree_body,
        out_type=jax.ShapeDtypeStruct((NC, HALF, D), jnp.float32),
        mesh=mesh,
        scratch_types=[
            pltpu.VMEM((NCH, CB), jnp.int32),
            pltpu.VMEM((CB, D), jnp.float32),
            pltpu.VMEM_SHARED((HALF, D), jnp.float32),
        ],
    )
    sc_scatter = pl.kernel(
        _sc_scatter_body,
        out_type=jax.ShapeDtypeStruct((NC, HALF, D), jnp.float32),
        mesh=mesh,
        scratch_types=[
            pltpu.VMEM((NCH, CB), jnp.int32),
            pltpu.VMEM((NCH, CB), jnp.int32),
            pltpu.VMEM((CB, D), jnp.float32),
            pltpu.VMEM((CB, D), jnp.float32),
            pltpu.VMEM((CB, D), jnp.float32),
            pltpu.VMEM_SHARED((HALF, D), jnp.float32),
        ] + [pltpu.SemaphoreType.DMA] * 6,
    )
    return sc_degree, sc_scatter


def _mm_body(x_ref, w_ref, o_ref):
    o_ref[...] = jnp.dot(x_ref[...], w_ref[...],
                         preferred_element_type=jnp.float32)


def _scale_body(h_ref, dg_ref, o_ref):
    deg = dg_ref[:, 0:1] + 1.0
    o_ref[...] = h_ref[...] * lax.rsqrt(deg)


def _mid_body(s_ref, g_ref, dg_ref, w_ref, b_ref, o_ref):
    dinv = lax.rsqrt(dg_ref[:, 0:1] + 1.0)
    z = dinv * (s_ref[...] + g_ref[...]) + b_ref[...]
    r = jnp.maximum(z, 0.0)
    o_ref[...] = jnp.dot(r, w_ref[...],
                         preferred_element_type=jnp.float32) * dinv


def _final_body(s_ref, g_ref, dg_ref, b_ref, o_ref):
    dinv = lax.rsqrt(dg_ref[:, 0:1] + 1.0)
    z = dinv * (s_ref[...] + g_ref[...]) + b_ref[...]
    m = jnp.max(z, axis=1, keepdims=True)
    e = jnp.exp(z - m)
    o_ref[...] = (z - m) - jnp.log(jnp.sum(e, axis=1, keepdims=True))


_row_spec = pl.BlockSpec((BR, D), lambda i: (i, 0))
_deg_spec = pl.BlockSpec((BR, D), lambda i: (i, 0))
_w_spec = pl.BlockSpec((D, D), lambda i: (0, 0))
_b_spec = pl.BlockSpec((1, D), lambda i: (0, 0))
_out_struct = jax.ShapeDtypeStruct((NPAD, D), jnp.float32)

_tc_mm = pl.pallas_call(
    _mm_body, grid=(GRID,),
    in_specs=[_row_spec, _w_spec], out_specs=_row_spec,
    out_shape=_out_struct)

_tc_scale = pl.pallas_call(
    _scale_body, grid=(GRID,),
    in_specs=[_row_spec, _deg_spec], out_specs=_row_spec,
    out_shape=_out_struct)

_tc_mid = pl.pallas_call(
    _mid_body, grid=(GRID,),
    in_specs=[_row_spec, _row_spec, _deg_spec, _w_spec, _b_spec],
    out_specs=_row_spec, out_shape=_out_struct)

_tc_final = pl.pallas_call(
    _final_body, grid=(GRID,),
    in_specs=[_row_spec, _row_spec, _deg_spec, _b_spec],
    out_specs=_row_spec, out_shape=_out_struct)


def kernel(x, edge_index, W1, b1, W2, b2):
    src = edge_index[0].astype(jnp.int32)
    dst = edge_index[1].astype(jnp.int32)

    fill = jnp.full((E_PAD - N_EDGES,), -1, jnp.int32)
    src_p = jnp.concatenate([src, fill])
    dst_p = jnp.concatenate([dst, fill])
    in_a = (dst_p >= 0) & (dst_p < HALF)
    in_b = dst_p >= HALF
    neg1 = jnp.int32(-1)

    def slab(a):
        return a.reshape(NCH, NS, CB).transpose(1, 0, 2)

    src_slabs = jnp.stack([slab(jnp.where(in_a, src_p, neg1)),
                           slab(jnp.where(in_b, src_p, neg1))])
    dst_slabs = jnp.stack([slab(jnp.where(in_a, dst_p, neg1)),
                           slab(jnp.where(in_b, dst_p - HALF, neg1))])
    src_slabs = src_slabs.reshape(NW, NCH, CB)
    dst_slabs = dst_slabs.reshape(NW, NCH, CB)

    x_pad = jnp.zeros((NPAD, D), jnp.float32).at[:N_NODES].set(x)
    b1r = b1.reshape(1, D)
    b2r = b2.reshape(1, D)

    sc_degree, sc_scatter = _sc_kernels()
    deg = sc_degree(dst_slabs).reshape(NPAD, D)

    h1 = _tc_mm(x_pad, W1)
    g1 = _tc_scale(h1, deg)
    s1 = sc_scatter(g1, src_slabs, dst_slabs).reshape(NPAD, D)
    g2 = _tc_mid(s1, g1, deg, W2, b1r)
    s2 = sc_scatter(g2, src_slabs, dst_slabs).reshape(NPAD, D)
    out = _tc_final(s2, g2, deg, b2r)
    return out[:N_NODES]


# speedup vs baseline: 22.3753x; 1.0108x over previous
import functools

import jax
import jax.numpy as jnp
from jax import lax
from jax.experimental import pallas as pl
from jax.experimental.pallas import tpu as pltpu
from jax.experimental.pallas import tpu_sc as plsc

N_NODES = 10000
N_EDGES = 320000
D = 128

NC = 2
NS = 16
NW = NC * NS

CB = 128
NCH = 159
NCHD = 162
E_PAD = NS * NCH * CB
E_PADD = NS * NCHD * CB
NPAD = 10240
HALF = NPAD // 2
RPT = HALF // NS

BR = 1024
GRID = NPAD // BR


def _sc_degree_body(dst_hbm, out_hbm, dst_v, buf_v, acc_s, d0, d1):
    dsem = [d0, d1]
    c = lax.axis_index("c")
    s = lax.axis_index("s")
    wid = c * NS + s
    base = s * RPT

    pltpu.sync_copy(dst_hbm.at[wid], dst_v)

    @pl.loop(0, 64)
    def _(i):
        for j in range(D // 16):
            buf_v[i, pl.ds(j * 16, 16)] = jnp.zeros((16,), jnp.float32)

    for k in range(RPT // 64):
        pltpu.sync_copy(buf_v.at[pl.ds(0, 64)],
                        acc_s.at[pl.ds(base + k * 64, 64)])
    plsc.subcore_barrier()

    @pl.loop(0, CB)
    def _(i):
        for j in range(D // 16):
            buf_v[i, pl.ds(j * 16, 16)] = jnp.full((16,), 1.0, jnp.float32)

    def _deg_scatter(j, sem):
        idx = plsc.Indices(dst_v.at[j], ignored_value=-1)
        pltpu.async_copy(buf_v, acc_s.at[idx], sem, add=True)

    def _deg_wait(j, sem):
        idx = plsc.Indices(dst_v.at[j], ignored_value=-1)
        pltpu.make_async_copy(buf_v, acc_s.at[idx], sem).wait()

    @pl.loop(0, NCHD // 2)
    def _(jj):
        for b in range(2):
            j = jj * 2 + b

            @pl.when(j >= 2)
            def _():
                _deg_wait(j - 2, dsem[b])

            _deg_scatter(j, dsem[b])

    for b in range(2):
        _deg_wait(NCHD - 2 + b, dsem[b])
    plsc.subcore_barrier()

    for k in range(RPT // 64):
        r0 = base + k * 64
        pltpu.sync_copy(acc_s.at[pl.ds(r0, 64)], buf_v.at[pl.ds(0, 64)])
        pltpu.sync_copy(buf_v.at[pl.ds(0, 64)], out_hbm.at[c].at[pl.ds(r0, 64)])


def _sc_scatter_body(g_hbm, src_hbm, dst_hbm, out_hbm,
                     src_v, dst_v, b0, b1, b2, acc_s,
                     g0, g1, g2, s0, s1, s2):
    c = lax.axis_index("c")
    s = lax.axis_index("s")
    wid = c * NS + s
    base = s * RPT
    bufs = [b0, b1, b2]
    gsem = [g0, g1, g2]
    ssem = [s0, s1, s2]

    pltpu.sync_copy(src_hbm.at[wid], src_v)
    pltpu.sync_copy(dst_hbm.at[wid], dst_v)

    def _gather(j, b):
        idx = plsc.Indices(src_v.at[j], ignored_value=-1)
        pltpu.async_copy(g_hbm.at[idx], bufs[b], gsem[b])

    def _gather_wait(j, b):
        idx = plsc.Indices(src_v.at[j], ignored_value=-1)
        pltpu.make_async_copy(g_hbm.at[idx], bufs[b], gsem[b]).wait()

    def _scatter(j, b):
        idx = plsc.Indices(dst_v.at[j], ignored_value=-1)
        pltpu.async_copy(bufs[b], acc_s.at[idx], ssem[b], add=True)

    def _scatter_wait(j, b):
        idx = plsc.Indices(dst_v.at[j], ignored_value=-1)
        pltpu.make_async_copy(bufs[b], acc_s.at[idx], ssem[b]).wait()

    for b in range(2):
        _gather(b, b)

    @pl.loop(0, 64)
    def _(i):
        for j in range(D // 16):
            b2[i, pl.ds(j * 16, 16)] = jnp.zeros((16,), jnp.float32)

    for k in range(RPT // 64):
        pltpu.sync_copy(b2.at[pl.ds(0, 64)],
                        acc_s.at[pl.ds(base + k * 64, 64)])
    plsc.subcore_barrier()

    @pl.loop(0, NCH // 3)
    def _(jj):
        for b in range(3):
            j = jj * 3 + b
            _gather_wait(j, b)
            _scatter(j, b)
            t = (b + 2) % 3
            jn = j + 2

            @pl.when(jn < NCH)
            def _():
                prev = jn - 3

                @pl.when(prev >= 0)
                def _():
                    _scatter_wait(prev, t)

                _gather(jn, t)

    for b in range(3):
        _scatter_wait(NCH - 3 + b, b)
    plsc.subcore_barrier()

    for k in range(RPT // 64):
        r0 = base + k * 64
        pltpu.sync_copy(acc_s.at[pl.ds(r0, 64)], b0.at[pl.ds(0, 64)])
        pltpu.sync_copy(b0.at[pl.ds(0, 64)], out_hbm.at[c].at[pl.ds(r0, 64)])


@functools.cache
def _sc_kernels():
    mesh = plsc.VectorSubcoreMesh(core_axis_name="c", subcore_axis_name="s",
                                  num_cores=NC, num_subcores=NS)
    sc_degree = pl.kernel(
        _sc_degree_body,
        out_type=jax.ShapeDtypeStruct((NC, HALF, D), jnp.float32),
        mesh=mesh,
        scratch_types=[
            pltpu.VMEM((NCHD, CB), jnp.int32),
            pltpu.VMEM((CB, D), jnp.float32),
            pltpu.VMEM_SHARED((HALF, D), jnp.float32),
            pltpu.SemaphoreType.DMA,
            pltpu.SemaphoreType.DMA,
        ],
    )
    sc_scatter = pl.kernel(
        _sc_scatter_body,
        out_type=jax.ShapeDtypeStruct((NC, HALF, D), jnp.float32),
        mesh=mesh,
        scratch_types=[
            pltpu.VMEM((NCH, CB), jnp.int32),
            pltpu.VMEM((NCH, CB), jnp.int32),
            pltpu.VMEM((CB, D), jnp.float32),
            pltpu.VMEM((CB, D), jnp.float32),
            pltpu.VMEM((CB, D), jnp.float32),
            pltpu.VMEM_SHARED((HALF, D), jnp.float32),
        ] + [pltpu.SemaphoreType.DMA] * 6,
    )
    return sc_degree, sc_scatter


def _mm_body(x_ref, w_ref, o_ref):
    o_ref[...] = jnp.dot(x_ref[...], w_ref[...],
                         preferred_element_type=jnp.float32)


def _scale_body(h_ref, dg_ref, o_ref):
    dinv = lax.rsqrt(dg_ref[:, 0:1] + 1.0)
    o_ref[...] = h_ref[...] * dinv


def _mid_body(s_ref, g_ref, dg_ref, w_ref, b_ref, o_ref):
    dinv = lax.rsqrt(dg_ref[:, 0:1] + 1.0)
    z = dinv * (s_ref[...] + g_ref[...]) + b_ref[...]
    r = jnp.maximum(z, 0.0)
    o_ref[...] = jnp.dot(r, w_ref[...],
                         preferred_element_type=jnp.float32) * dinv


def _final_body(s_ref, g_ref, dg_ref, b_ref, o_ref):
    dinv = lax.rsqrt(dg_ref[:, 0:1] + 1.0)
    z = dinv * (s_ref[...] + g_ref[...]) + b_ref[...]
    m = jnp.max(z, axis=1, keepdims=True)
    e = jnp.exp(z - m)
    o_ref[...] = (z - m) - jnp.log(jnp.sum(e, axis=1, keepdims=True))


_row_spec = pl.BlockSpec((BR, D), lambda i: (i, 0))
_deg_spec = pl.BlockSpec((BR, D), lambda i: (i, 0))
_w_spec = pl.BlockSpec((D, D), lambda i: (0, 0))
_b_spec = pl.BlockSpec((1, D), lambda i: (0, 0))
_out_struct = jax.ShapeDtypeStruct((NPAD, D), jnp.float32)

_tc_mm = pl.pallas_call(
    _mm_body, grid=(GRID,),
    in_specs=[_row_spec, _w_spec], out_specs=_row_spec,
    out_shape=_out_struct)

_tc_scale = pl.pallas_call(
    _scale_body, grid=(GRID,),
    in_specs=[_row_spec, _deg_spec], out_specs=_row_spec,
    out_shape=_out_struct)

_tc_mid = pl.pallas_call(
    _mid_body, grid=(GRID,),
    in_specs=[_row_spec, _row_spec, _deg_spec, _w_spec, _b_spec],
    out_specs=_row_spec, out_shape=_out_struct)

_tc_final = pl.pallas_call(
    _final_body, grid=(GRID,),
    in_specs=[_row_spec, _row_spec, _deg_spec, _b_spec],
    out_specs=_row_spec, out_shape=_out_struct)


def kernel(x, edge_index, W1, b1, W2, b2):
    src = edge_index[0].astype(jnp.int32)
    dst = edge_index[1].astype(jnp.int32)

    neg1 = jnp.int32(-1)

    def slabs_for(nch, arr_src, arr_dst):
        e_pad = NS * nch * CB
        fill = jnp.full((e_pad - N_EDGES,), -1, jnp.int32)
        src_p = jnp.concatenate([arr_src, fill])
        dst_p = jnp.concatenate([arr_dst, fill])
        in_a = (dst_p >= 0) & (dst_p < HALF)
        in_b = dst_p >= HALF

        def slab(a):
            return a.reshape(nch, NS, CB).transpose(1, 0, 2)

        s_sl = jnp.stack([slab(jnp.where(in_a, src_p, neg1)),
                          slab(jnp.where(in_b, src_p, neg1))])
        d_sl = jnp.stack([slab(jnp.where(in_a, dst_p, neg1)),
                          slab(jnp.where(in_b, dst_p - HALF, neg1))])
        return s_sl.reshape(NW, nch, CB), d_sl.reshape(NW, nch, CB)

    src_slabs, dst_slabs = slabs_for(NCH, src, dst)
    _, dst_slabs_d = slabs_for(NCHD, src, dst)

    x_pad = jnp.zeros((NPAD, D), jnp.float32).at[:N_NODES].set(x)
    b1r = b1.reshape(1, D)
    b2r = b2.reshape(1, D)

    sc_degree, sc_scatter = _sc_kernels()
    deg = sc_degree(dst_slabs_d).reshape(NPAD, D)

    h1 = _tc_mm(x_pad, W1)
    g1 = _tc_scale(h1, deg)
    s1 = sc_scatter(g1, src_slabs, dst_slabs).reshape(NPAD, D)
    g2 = _tc_mid(s1, g1, deg, W2, b1r)
    s2 = sc_scatter(g2, src_slabs, dst_slabs).reshape(NPAD, D)
    out = _tc_final(s2, g2, deg, b2r)
    return out[:N_NODES]


# speedup vs baseline: 22.6032x; 1.0102x over previous
import functools

import jax
import jax.numpy as jnp
from jax import lax
from jax.experimental import pallas as pl
from jax.experimental.pallas import tpu as pltpu
from jax.experimental.pallas import tpu_sc as plsc

N_NODES = 10000
N_EDGES = 320000
D = 128

NC = 2
NS = 16
NW = NC * NS

CB = 128
NCH = 159
E_PAD = NS * NCH * CB
NPAD = 10240
HALF = NPAD // 2
RPT = HALF // NS

BR = 1024
GRID = NPAD // BR


def _sc_degree_body(dst_hbm, out_hbm, dst_v, buf_v, acc_s, d0, d1, d2, d3):
    dsem = [d0, d1, d2, d3]
    c = lax.axis_index("c")
    s = lax.axis_index("s")
    wid = c * NS + s
    base = s * RPT

    pltpu.sync_copy(dst_hbm.at[wid], dst_v)

    @pl.loop(0, 64)
    def _(i):
        for j in range(D // 16):
            buf_v[i, pl.ds(j * 16, 16)] = jnp.zeros((16,), jnp.float32)

    for k in range(RPT // 64):
        pltpu.sync_copy(buf_v.at[pl.ds(0, 64)],
                        acc_s.at[pl.ds(base + k * 64, 64)])
    plsc.subcore_barrier()

    @pl.loop(0, CB)
    def _(i):
        for j in range(D // 16):
            buf_v[i, pl.ds(j * 16, 16)] = jnp.full((16,), 1.0, jnp.float32)

    def _deg_scatter(j, sem):
        idx = plsc.Indices(dst_v.at[j], ignored_value=-1)
        pltpu.async_copy(buf_v, acc_s.at[idx], sem, add=True)

    def _deg_wait(j, sem):
        idx = plsc.Indices(dst_v.at[j], ignored_value=-1)
        pltpu.make_async_copy(buf_v, acc_s.at[idx], sem).wait()

    nfull = (NCH // 4) * 4

    @pl.loop(0, NCH // 4)
    def _(jj):
        for b in range(4):
            j = jj * 4 + b

            @pl.when(j >= 4)
            def _():
                _deg_wait(j - 4, dsem[b])

            _deg_scatter(j, dsem[b])

    for j in range(nfull, NCH):
        _deg_wait(j - 4, dsem[j % 4])
        _deg_scatter(j, dsem[j % 4])

    for j in range(NCH - 4, NCH):
        _deg_wait(j, dsem[j % 4])
    plsc.subcore_barrier()

    for k in range(RPT // 64):
        r0 = base + k * 64
        pltpu.sync_copy(acc_s.at[pl.ds(r0, 64)], buf_v.at[pl.ds(0, 64)])
        pltpu.sync_copy(buf_v.at[pl.ds(0, 64)], out_hbm.at[c].at[pl.ds(r0, 64)])


def _sc_scatter_body(g_hbm, src_hbm, dst_hbm, out_hbm,
                     src_v, dst_v, b0, b1, b2, acc_s,
                     g0, g1, g2, s0, s1, s2):
    c = lax.axis_index("c")
    s = lax.axis_index("s")
    wid = c * NS + s
    base = s * RPT
    bufs = [b0, b1, b2]
    gsem = [g0, g1, g2]
    ssem = [s0, s1, s2]

    pltpu.sync_copy(src_hbm.at[wid], src_v)
    pltpu.sync_copy(dst_hbm.at[wid], dst_v)

    def _gather(j, b):
        idx = plsc.Indices(src_v.at[j], ignored_value=-1)
        pltpu.async_copy(g_hbm.at[idx], bufs[b], gsem[b])

    def _gather_wait(j, b):
        idx = plsc.Indices(src_v.at[j], ignored_value=-1)
        pltpu.make_async_copy(g_hbm.at[idx], bufs[b], gsem[b]).wait()

    def _scatter(j, b):
        idx = plsc.Indices(dst_v.at[j], ignored_value=-1)
        pltpu.async_copy(bufs[b], acc_s.at[idx], ssem[b], add=True)

    def _scatter_wait(j, b):
        idx = plsc.Indices(dst_v.at[j], ignored_value=-1)
        pltpu.make_async_copy(bufs[b], acc_s.at[idx], ssem[b]).wait()

    for b in range(2):
        _gather(b, b)

    @pl.loop(0, 64)
    def _(i):
        for j in range(D // 16):
            b2[i, pl.ds(j * 16, 16)] = jnp.zeros((16,), jnp.float32)

    for k in range(RPT // 64):
        pltpu.sync_copy(b2.at[pl.ds(0, 64)],
                        acc_s.at[pl.ds(base + k * 64, 64)])
    plsc.subcore_barrier()

    @pl.loop(0, NCH // 3)
    def _(jj):
        for b in range(3):
            j = jj * 3 + b
            _gather_wait(j, b)
            _scatter(j, b)
            t = (b + 2) % 3
            jn = j + 2

            @pl.when(jn < NCH)
            def _():
                prev = jn - 3

                @pl.when(prev >= 0)
                def _():
                    _scatter_wait(prev, t)

                _gather(jn, t)

    for b in range(3):
        _scatter_wait(NCH - 3 + b, b)
    plsc.subcore_barrier()

    for k in range(RPT // 64):
        r0 = base + k * 64
        pltpu.sync_copy(acc_s.at[pl.ds(r0, 64)], b0.at[pl.ds(0, 64)])
        pltpu.sync_copy(b0.at[pl.ds(0, 64)], out_hbm.at[c].at[pl.ds(r0, 64)])


@functools.cache
def _sc_kernels():
    mesh = plsc.VectorSubcoreMesh(core_axis_name="c", subcore_axis_name="s",
                                  num_cores=NC, num_subcores=NS)
    sc_degree = pl.kernel(
        _sc_degree_body,
        out_type=jax.ShapeDtypeStruct((NC, HALF, D), jnp.float32),
        mesh=mesh,
        scratch_types=[
            pltpu.VMEM((NCH, CB), jnp.int32),
            pltpu.VMEM((CB, D), jnp.float32),
            pltpu.VMEM_SHARED((HALF, D), jnp.float32),
        ] + [pltpu.SemaphoreType.DMA] * 4,
    )
    sc_scatter = pl.kernel(
        _sc_scatter_body,
        out_type=jax.ShapeDtypeStruct((NC, HALF, D), jnp.float32),
        mesh=mesh,
        scratch_types=[
            pltpu.VMEM((NCH, CB), jnp.int32),
            pltpu.VMEM((NCH, CB), jnp.int32),
            pltpu.VMEM((CB, D), jnp.float32),
            pltpu.VMEM((CB, D), jnp.float32),
            pltpu.VMEM((CB, D), jnp.float32),
            pltpu.VMEM_SHARED((HALF, D), jnp.float32),
        ] + [pltpu.SemaphoreType.DMA] * 6,
    )
    return sc_degree, sc_scatter


def _mm_scale_body(x_ref, w_ref, dg_ref, o_ref):
    dinv = lax.rsqrt(dg_ref[:, 0:1] + 1.0)
    o_ref[...] = jnp.dot(x_ref[...], w_ref[...],
                         preferred_element_type=jnp.float32) * dinv


def _mid_body(s_ref, g_ref, dg_ref, w_ref, b_ref, o_ref):
    dinv = lax.rsqrt(dg_ref[:, 0:1] + 1.0)
    z = dinv * (s_ref[...] + g_ref[...]) + b_ref[...]
    r = jnp.maximum(z, 0.0)
    o_ref[...] = jnp.dot(r, w_ref[...],
                         preferred_element_type=jnp.float32) * dinv


def _final_body(s_ref, g_ref, dg_ref, b_ref, o_ref):
    dinv = lax.rsqrt(dg_ref[:, 0:1] + 1.0)
    z = dinv * (s_ref[...] + g_ref[...]) + b_ref[...]
    m = jnp.max(z, axis=1, keepdims=True)
    e = jnp.exp(z - m)
    o_ref[...] = (z - m) - jnp.log(jnp.sum(e, axis=1, keepdims=True))


_row_spec = pl.BlockSpec((BR, D), lambda i: (i, 0))
_deg_spec = pl.BlockSpec((BR, D), lambda i: (i, 0))
_w_spec = pl.BlockSpec((D, D), lambda i: (0, 0))
_b_spec = pl.BlockSpec((1, D), lambda i: (0, 0))
_out_struct = jax.ShapeDtypeStruct((NPAD, D), jnp.float32)

_tc_mm_scale = pl.pallas_call(
    _mm_scale_body, grid=(GRID,),
    in_specs=[_row_spec, _w_spec, _deg_spec], out_specs=_row_spec,
    out_shape=_out_struct)

_tc_mid = pl.pallas_call(
    _mid_body, grid=(GRID,),
    in_specs=[_row_spec, _row_spec, _deg_spec, _w_spec, _b_spec],
    out_specs=_row_spec, out_shape=_out_struct)

_tc_final = pl.pallas_call(
    _final_body, grid=(GRID,),
    in_specs=[_row_spec, _row_spec, _deg_spec, _b_spec],
    out_specs=_row_spec, out_shape=_out_struct)


def kernel(x, edge_index, W1, b1, W2, b2):
    src = edge_index[0].astype(jnp.int32)
    dst = edge_index[1].astype(jnp.int32)

    neg1 = jnp.int32(-1)

    fill = jnp.full((E_PAD - N_EDGES,), -1, jnp.int32)
    src_p = jnp.concatenate([src, fill])
    dst_p = jnp.concatenate([dst, fill])
    in_a = (dst_p >= 0) & (dst_p < HALF)
    in_b = dst_p >= HALF

    def slab(a):
        return a.reshape(NCH, NS, CB).transpose(1, 0, 2)

    s_sl = jnp.stack([slab(jnp.where(in_a, src_p, neg1)),
                      slab(jnp.where(in_b, src_p, neg1))])
    d_sl = jnp.stack([slab(jnp.where(in_a, dst_p, neg1)),
                      slab(jnp.where(in_b, dst_p - HALF, neg1))])
    src_slabs = s_sl.reshape(NW, NCH, CB)
    dst_slabs = d_sl.reshape(NW, NCH, CB)

    x_pad = jnp.zeros((NPAD, D), jnp.float32).at[:N_NODES].set(x)
    b1r = b1.reshape(1, D)
    b2r = b2.reshape(1, D)

    sc_degree, sc_scatter = _sc_kernels()
    deg = sc_degree(dst_slabs).reshape(NPAD, D)

    g1 = _tc_mm_scale(x_pad, W1, deg)
    s1 = sc_scatter(g1, src_slabs, dst_slabs).reshape(NPAD, D)
    g2 = _tc_mid(s1, g1, deg, W2, b1r)
    s2 = sc_scatter(g2, src_slabs, dst_slabs).reshape(NPAD, D)
    out = _tc_final(s2, g2, deg, b2r)
    return out[:N_NODES]
